# baseline (speedup 1.0000x reference)
import functools
import math

import jax
import jax.numpy as jnp
from jax import lax
from jax.experimental import pallas as pl
from jax.experimental.pallas import tpu as pltpu
from jax.experimental.pallas import tpu_sc as plsc

D_MODEL = 128
SCALE = math.sqrt(float(D_MODEL))

NC = 2
NS = 16
NW = NC * NS

CHUNK = 128
NBUF = 4
LANES = 16
PACKED_W = D_MODEL // 2


def _pack_table(lut):
    v, d = lut.shape
    block = 2000
    assert v % block == 0 and d == D_MODEL

    def rnd(u):
        return (u + jnp.uint32(0x7FFF) + ((u >> 16) & jnp.uint32(1))) >> 16

    def body(l_ref, o_ref):
        x = l_ref[...] * SCALE
        bits = jax.lax.bitcast_convert_type(x, jnp.uint32)
        for g in range(4):
            lo = rnd(bits[:, g * 32:g * 32 + 16])
            hi = rnd(bits[:, g * 32 + 16:g * 32 + 32])
            o_ref[:, g * 16:(g + 1) * 16] = jax.lax.bitcast_convert_type((hi << 16) | lo, jnp.int32)

    return pl.pallas_call(
        body,
        grid=(v // block,),
        in_specs=[pl.BlockSpec((block, d), lambda i: (i, 0))],
        out_specs=pl.BlockSpec((block, PACKED_W), lambda i: (i, 0)),
        out_shape=jax.ShapeDtypeStruct((v, PACKED_W), jnp.int32),
    )(lut)


def _make_gather(n_idx):
    assert n_idx % (NW * CHUNK * NBUF) == 0
    steps = n_idx // (NW * CHUNK)
    ngroups = steps // NBUF
    mesh = plsc.VectorSubcoreMesh(
        core_axis_name="c", subcore_axis_name="s",
        num_cores=NC, num_subcores=NS)

    @functools.partial(
        pl.kernel,
        out_type=jax.ShapeDtypeStruct((n_idx, D_MODEL), jnp.float32),
        mesh=mesh,
        compiler_params=pltpu.CompilerParams(use_tc_tiling_on_sc=False),
        scratch_types=(
            [pltpu.VMEM((steps, CHUNK), jnp.int32)]
            + [pltpu.VMEM((CHUNK, PACKED_W), jnp.int32)] * NBUF
            + [pltpu.VMEM((CHUNK, D_MODEL), jnp.float32)] * NBUF
            + [pltpu.SemaphoreType.DMA] * (2 * NBUF + 1)
        ),
    )
    def gather(idx_hbm, table_hbm, out_hbm, idx_v, *bufs_and_sems):
        inb = bufs_and_sems[:NBUF]
        outb = bufs_and_sems[NBUF:2 * NBUF]
        gsem = bufs_and_sems[2 * NBUF:3 * NBUF]
        ssem = bufs_and_sems[3 * NBUF:4 * NBUF]
        isem = bufs_and_sems[4 * NBUF]
        wid = lax.axis_index("s") * NC + lax.axis_index("c")
        base = wid * steps

        pltpu.async_copy(idx_hbm.at[pl.ds(base, steps)], idx_v, isem).wait()

        def fire_gather(j, b):
            pltpu.async_copy(table_hbm.at[idx_v.at[j]], inb[b], gsem[b])

        def fire_store(j, b):
            pltpu.async_copy(
                outb[b], out_hbm.at[pl.ds((base + j) * CHUNK, CHUNK)],
                ssem[b])

        def wait_gather(j, b):
            pltpu.make_async_copy(
                table_hbm.at[idx_v.at[j]], inb[b], gsem[b]).wait()

        def wait_store(j, b):
            pltpu.make_async_copy(
                outb[b], out_hbm.at[pl.ds((base + j) * CHUNK, CHUNK)],
                ssem[b]).wait()

        def expand_buf(b):
            src, dst = inb[b], outb[b]

            @plsc.parallel_loop(0, CHUNK, 1, unroll=2)
            def _(r):
                for u in range(4):
                    w = src[r, pl.ds(u * 16, 16)]
                    lo = jax.lax.bitcast_convert_type(
                        w << 16, jnp.float32)
                    hi = jax.lax.bitcast_convert_type(
                        w & jnp.int32(-65536), jnp.float32)
                    dst[r, pl.ds(u * 32, LANES)] = lo
                    dst[r, pl.ds(u * 32 + 16, LANES)] = hi

        for b in range(NBUF):
            fire_gather(b, b)

        def group(g, carry):
            j0 = g * NBUF
            for b in range(NBUF):
                wait_gather(j0 + b, b)
                expand_buf(b)
                fire_store(j0 + b, b)

            @pl.when(g < ngroups - 1)
            def _():
                for b in range(NBUF):
                    wait_store(j0 + b, b)
                    fire_gather(j0 + NBUF + b, b)

            return carry

        lax.fori_loop(0, ngroups, group, 0)
        j_last = (ngroups - 1) * NBUF
        for b in range(NBUF):
            wait_store(j_last + b, b)

    return gather


def kernel(x, lut):
    b0, b1 = x.shape
    n_idx = b0 * b1
    idx = x.reshape(n_idx // CHUNK, CHUNK).astype(jnp.int32)
    table = _pack_table(lut)
    out = _make_gather(n_idx)(idx, table)
    return out.reshape(b0, b1, D_MODEL)


# speedup vs baseline: 1.1783x; 1.1783x over previous
import functools
import math

import jax
import jax.numpy as jnp
from jax import lax
from jax.experimental import pallas as pl
from jax.experimental.pallas import tpu as pltpu
from jax.experimental.pallas import tpu_sc as plsc

D_MODEL = 128
SCALE = math.sqrt(float(D_MODEL))

NC = 2
NS = 16
NW = NC * NS

CHUNK = 128
NBUF = 5
LANES = 16


def _make_gather(n_idx):
    assert n_idx % (NW * CHUNK * NBUF) == 0
    steps = n_idx // (NW * CHUNK)
    ngroups = steps // NBUF
    vecs_per_row = D_MODEL // LANES
    mesh = plsc.VectorSubcoreMesh(
        core_axis_name="c", subcore_axis_name="s",
        num_cores=NC, num_subcores=NS)

    @functools.partial(
        pl.kernel,
        out_type=jax.ShapeDtypeStruct((n_idx, D_MODEL), jnp.float32),
        mesh=mesh,
        scratch_types=(
            [pltpu.VMEM((steps, CHUNK), jnp.int32)]
            + [pltpu.VMEM((CHUNK, D_MODEL), jnp.float32)] * NBUF
            + [pltpu.SemaphoreType.DMA] * (2 * NBUF + 1)
        ),
    )
    def gather(idx_hbm, table_hbm, out_hbm, idx_v, *bufs_and_sems):
        rows = bufs_and_sems[:NBUF]
        gsem = bufs_and_sems[NBUF:2 * NBUF]
        ssem = bufs_and_sems[2 * NBUF:3 * NBUF]
        isem = bufs_and_sems[3 * NBUF]
        wid = lax.axis_index("s") * NC + lax.axis_index("c")
        base = wid * steps

        pltpu.async_copy(idx_hbm.at[pl.ds(base, steps)], idx_v, isem).wait()

        def fire_gather(j, b):
            pltpu.async_copy(table_hbm.at[idx_v.at[j]], rows[b], gsem[b])

        def fire_store(j, b):
            pltpu.async_copy(
                rows[b], out_hbm.at[pl.ds((base + j) * CHUNK, CHUNK)], ssem[b])

        def wait_gather(j, b):
            pltpu.make_async_copy(
                table_hbm.at[idx_v.at[j]], rows[b], gsem[b]).wait()

        def wait_store(j, b):
            pltpu.make_async_copy(
                rows[b], out_hbm.at[pl.ds((base + j) * CHUNK, CHUNK)],
                ssem[b]).wait()

        def scale_buf(b):
            buf = rows[b]

            @plsc.parallel_loop(0, CHUNK, 1, unroll=4)
            def _(r):
                for u in range(vecs_per_row):
                    sl = pl.ds(u * LANES, LANES)
                    buf[r, sl] = buf[r, sl] * SCALE

        for b in range(NBUF):
            fire_gather(b, b)

        def group(g, carry):
            j0 = g * NBUF
            for b in range(NBUF):
                wait_gather(j0 + b, b)
                scale_buf(b)
                fire_store(j0 + b, b)

            @pl.when(g < ngroups - 1)
            def _():
                for b in range(NBUF):
                    wait_store(j0 + b, b)
                    fire_gather(j0 + NBUF + b, b)

            return carry

        lax.fori_loop(0, ngroups, group, 0)
        j_last = (ngroups - 1) * NBUF
        for b in range(NBUF):
            wait_store(j_last + b, b)

    return gather


def kernel(x, lut):
    b0, b1 = x.shape
    n_idx = b0 * b1
    idx = x.reshape(n_idx // CHUNK, CHUNK).astype(jnp.int32)
    out = _make_gather(n_idx)(idx, lut)
    return out.reshape(b0, b1, D_MODEL)
